# baseline (speedup 1.0000x reference)
import jax
import jax.numpy as jnp
from jax import lax
from jax.experimental import pallas as pl
from jax.experimental.pallas import tpu as pltpu
from jax.experimental.pallas import tpu_sc as plsc

N_NODES = 10000
N_EDGES = 320000
D = 128

NC = 2
NS = 16
NW = NC * NS

CHUNK = 80
SB = 6
NSB = 22
NCHUNK = SB * NSB
E_PAD = NW * NCHUNK * CHUNK
NPAD = 10112
ROWS_PER_TILE = NPAD // NS


def _scatter_gather_kernel(x_hbm, pk_hbm, zero_hbm, h2_hbm,
                           pk0, pk1, sidx0, sidx1, sidx2,
                           didx0, didx1, didx2, rows0, rows1, rows2, acc,
                           gsem0, gsem1, gsem2, ssem0, ssem1, ssem2,
                           psem0, psem1):
    c = lax.axis_index("c")
    s = lax.axis_index("s")
    wid = s * NC + c

    pk = (pk0, pk1)
    sidx = (sidx0, sidx1, sidx2)
    didx = (didx0, didx1, didx2)
    bufs = (rows0, rows1, rows2)
    gsem = (gsem0, gsem1, gsem2)
    ssem = (ssem0, ssem1, ssem2)
    psem = (psem0, psem1)

    def unpack(slot, row, r):
        for t in range(CHUNK // 16):
            v = pk[slot][row, pl.ds(t * 16, 16)]
            sidx[r][pl.ds(t * 16, 16)] = jnp.bitwise_and(v, 0xFFFF)
            didx[r][pl.ds(t * 16, 16)] = lax.shift_right_logical(v, 16)

    def gather(r):
        pltpu.async_copy(x_hbm.at[sidx[r]], bufs[r], gsem[r])

    def gather_wait(r):
        pltpu.make_async_copy(x_hbm.at[sidx[r]], bufs[r], gsem[r]).wait()

    def scatter(r):
        pltpu.async_copy(bufs[r], acc.at[didx[r]], ssem[r], add=True)

    def scatter_wait(r):
        pltpu.make_async_copy(bufs[r], acc.at[didx[r]], ssem[r]).wait()

    def prefetch(sb_next, slot):
        pltpu.async_copy(pk_hbm.at[wid, sb_next], pk[slot], psem[slot])

    def prefetch_wait(slot):
        pltpu.make_async_copy(pk_hbm.at[wid, 0], pk[slot], psem[slot]).wait()

    def step(k, sbk, drain_prev=True):
        r = k % 3
        rn = (k + 2) % 3
        gather_wait(r)
        scatter(r)
        if drain_prev:
            scatter_wait(rn)
        if k == SB - 2:
            prefetch_wait(1 - sbk)
        if k < SB - 2:
            unpack(sbk, k + 2, rn)
        else:
            unpack(1 - sbk, k + 2 - SB, rn)
        gather(rn)
        return None

    r0 = s * ROWS_PER_TILE
    pltpu.sync_copy(zero_hbm.at[pl.ds(r0, ROWS_PER_TILE)],
                    acc.at[pl.ds(r0, ROWS_PER_TILE)])

    pltpu.sync_copy(pk_hbm.at[wid, 0], pk0)
    plsc.subcore_barrier()
    prefetch(1, 1)
    unpack(0, 0, 0)
    unpack(0, 1, 1)
    gather(0)
    gather(1)

    step(0, 0, drain_prev=False)
    for k in range(1, SB):
        step(k, 0)
    prefetch(2, 0)
    for k in range(SB):
        step(k, 1)

    def body(i, carry):
        for sbk in range(2):
            sb = 2 + 2 * i + sbk
            sbn = lax.rem(sb + 1, NSB)
            prefetch(sbn, 1 - sbk)
            for k in range(SB):
                step(k, sbk)
        return carry

    lax.fori_loop(0, (NSB - 2) // 2, body, 0)

    scatter_wait((NCHUNK - 1) % 3)
    gather_wait(NCHUNK % 3)
    gather_wait((NCHUNK + 1) % 3)

    plsc.subcore_barrier()
    pltpu.sync_copy(acc.at[pl.ds(r0, ROWS_PER_TILE)],
                    h2_hbm.at[c, pl.ds(r0, ROWS_PER_TILE)])


@jax.jit
def _segment_sum_sc(x, pk, zero):
    mesh = plsc.VectorSubcoreMesh(core_axis_name="c", subcore_axis_name="s")
    return pl.kernel(
        _scatter_gather_kernel,
        out_type=jax.ShapeDtypeStruct((NC, NPAD, D), jnp.float32),
        mesh=mesh,
        scratch_types=[
            pltpu.VMEM((SB, CHUNK), jnp.int32),
            pltpu.VMEM((SB, CHUNK), jnp.int32),
            pltpu.VMEM((CHUNK,), jnp.int32),
            pltpu.VMEM((CHUNK,), jnp.int32),
            pltpu.VMEM((CHUNK,), jnp.int32),
            pltpu.VMEM((CHUNK,), jnp.int32),
            pltpu.VMEM((CHUNK,), jnp.int32),
            pltpu.VMEM((CHUNK,), jnp.int32),
            pltpu.VMEM((CHUNK, D), jnp.float32),
            pltpu.VMEM((CHUNK, D), jnp.float32),
            pltpu.VMEM((CHUNK, D), jnp.float32),
            pltpu.VMEM_SHARED((NPAD, D), jnp.float32),
        ] + [pltpu.SemaphoreType.DMA] * 8,
    )(x, pk, zero)


def _linear_body(h2_ref, w_ref, b_ref, o_ref):
    h = h2_ref[0] + h2_ref[1]
    o_ref[...] = lax.dot_general(
        h, w_ref[...], (((1,), (1,)), ((), ())),
        preferred_element_type=jnp.float32) + b_ref[...]


@jax.jit
def _linear_tc(h2, W, b2):
    blk = 1000
    grid = N_NODES // blk
    return pl.pallas_call(
        _linear_body,
        grid=(grid,),
        in_specs=[
            pl.BlockSpec((NC, blk, D), lambda i: (0, i, 0)),
            pl.BlockSpec((D, D), lambda i: (0, 0)),
            pl.BlockSpec((1, D), lambda i: (0, 0)),
        ],
        out_specs=pl.BlockSpec((blk, D), lambda i: (i, 0)),
        out_shape=jax.ShapeDtypeStruct((N_NODES, D), jnp.float32),
    )(h2, W, b2)


def kernel(inputs, edge_index, W, b):
    n_pad = E_PAD - N_EDGES
    src = jnp.concatenate(
        [edge_index[0], jnp.zeros((n_pad,), jnp.int32)])
    pad_dst = N_NODES + jnp.arange(n_pad, dtype=jnp.int32) % (NPAD - N_NODES)
    dst = jnp.concatenate([edge_index[1], pad_dst])
    pk = (src | (dst << 16)).reshape(NW, NSB, SB, CHUNK)
    zero = jnp.zeros((NPAD, D), jnp.float32)
    h2 = _segment_sum_sc(inputs, pk, zero)
    return _linear_tc(h2, W, b.reshape(1, D))


# speedup vs baseline: 1.0001x; 1.0001x over previous
import jax
import jax.numpy as jnp
from jax import lax
from jax.experimental import pallas as pl
from jax.experimental.pallas import tpu as pltpu
from jax.experimental.pallas import tpu_sc as plsc

N_NODES = 10000
N_EDGES = 320000
D = 128

NC = 2
NS = 16
NW = NC * NS

CHUNK = 80
SB = 6
NSB = 22
NCHUNK = SB * NSB
E_PAD = NW * NCHUNK * CHUNK
NPAD = 10112
ROWS_PER_TILE = NPAD // NS


def _scatter_gather_kernel(x_hbm, pk_hbm, zero_hbm, h2_hbm,
                           pk0, pk1, sidx0, sidx1, sidx2,
                           didx0, didx1, didx2, rows0, rows1, rows2, acc,
                           gsem0, gsem1, gsem2, psem0, psem1):
    c = lax.axis_index("c")
    s = lax.axis_index("s")
    wid = s * NC + c

    pk = (pk0, pk1)
    sidx = (sidx0, sidx1, sidx2)
    didx = (didx0, didx1, didx2)
    bufs = (rows0, rows1, rows2)
    gsem = (gsem0, gsem1, gsem2)
    psem = (psem0, psem1)

    def unpack(slot, row, r):
        for t in range(CHUNK // 16):
            v = pk[slot][row, pl.ds(t * 16, 16)]
            sidx[r][pl.ds(t * 16, 16)] = jnp.bitwise_and(v, 0xFFFF)
            didx[r][pl.ds(t * 16, 16)] = lax.shift_right_logical(v, 16)

    def gather(r):
        pltpu.async_copy(x_hbm.at[sidx[r]], bufs[r], gsem[r])

    def gather_wait(r):
        pltpu.make_async_copy(x_hbm.at[sidx[r]], bufs[r], gsem[r]).wait()

    def scatter(r):
        pltpu.sync_copy(bufs[r], acc.at[didx[r]], add=True)

    def prefetch(sb_next, slot):
        pltpu.async_copy(pk_hbm.at[wid, sb_next], pk[slot], psem[slot])

    def prefetch_wait(slot):
        pltpu.make_async_copy(pk_hbm.at[wid, 0], pk[slot], psem[slot]).wait()

    def step(k, sbk):
        r = k % 3
        rn = (k + 2) % 3
        gather_wait(r)
        if k == SB - 2:
            prefetch_wait(1 - sbk)
        if k < SB - 2:
            unpack(sbk, k + 2, rn)
        else:
            unpack(1 - sbk, k + 2 - SB, rn)
        gather(rn)
        scatter(r)
        return None

    r0 = s * ROWS_PER_TILE
    pltpu.sync_copy(zero_hbm.at[pl.ds(r0, ROWS_PER_TILE)],
                    acc.at[pl.ds(r0, ROWS_PER_TILE)])

    pltpu.sync_copy(pk_hbm.at[wid, 0], pk0)
    plsc.subcore_barrier()
    prefetch(1, 1)
    unpack(0, 0, 0)
    unpack(0, 1, 1)
    gather(0)
    gather(1)

    for k in range(SB):
        step(k, 0)
    prefetch(2, 0)
    for k in range(SB):
        step(k, 1)

    def body(i, carry):
        for sbk in range(2):
            sb = 2 + 2 * i + sbk
            sbn = lax.rem(sb + 1, NSB)
            prefetch(sbn, 1 - sbk)
            for k in range(SB):
                step(k, sbk)
        return carry

    lax.fori_loop(0, (NSB - 2) // 2, body, 0)

    gather_wait(NCHUNK % 3)
    gather_wait((NCHUNK + 1) % 3)

    plsc.subcore_barrier()
    pltpu.sync_copy(acc.at[pl.ds(r0, ROWS_PER_TILE)],
                    h2_hbm.at[c, pl.ds(r0, ROWS_PER_TILE)])


@jax.jit
def _segment_sum_sc(x, pk, zero):
    mesh = plsc.VectorSubcoreMesh(core_axis_name="c", subcore_axis_name="s")
    return pl.kernel(
        _scatter_gather_kernel,
        out_type=jax.ShapeDtypeStruct((NC, NPAD, D), jnp.float32),
        mesh=mesh,
        scratch_types=[
            pltpu.VMEM((SB, CHUNK), jnp.int32),
            pltpu.VMEM((SB, CHUNK), jnp.int32),
            pltpu.VMEM((CHUNK,), jnp.int32),
            pltpu.VMEM((CHUNK,), jnp.int32),
            pltpu.VMEM((CHUNK,), jnp.int32),
            pltpu.VMEM((CHUNK,), jnp.int32),
            pltpu.VMEM((CHUNK,), jnp.int32),
            pltpu.VMEM((CHUNK,), jnp.int32),
            pltpu.VMEM((CHUNK, D), jnp.float32),
            pltpu.VMEM((CHUNK, D), jnp.float32),
            pltpu.VMEM((CHUNK, D), jnp.float32),
            pltpu.VMEM_SHARED((NPAD, D), jnp.float32),
        ] + [pltpu.SemaphoreType.DMA] * 5,
    )(x, pk, zero)


def _linear_body(h2_ref, w_ref, b_ref, o_ref):
    h = h2_ref[0] + h2_ref[1]
    o_ref[...] = lax.dot_general(
        h, w_ref[...], (((1,), (1,)), ((), ())),
        preferred_element_type=jnp.float32) + b_ref[...]


@jax.jit
def _linear_tc(h2, W, b2):
    blk = 1000
    grid = N_NODES // blk
    return pl.pallas_call(
        _linear_body,
        grid=(grid,),
        in_specs=[
            pl.BlockSpec((NC, blk, D), lambda i: (0, i, 0)),
            pl.BlockSpec((D, D), lambda i: (0, 0)),
            pl.BlockSpec((1, D), lambda i: (0, 0)),
        ],
        out_specs=pl.BlockSpec((blk, D), lambda i: (i, 0)),
        out_shape=jax.ShapeDtypeStruct((N_NODES, D), jnp.float32),
    )(h2, W, b2)


def kernel(inputs, edge_index, W, b):
    n_pad = E_PAD - N_EDGES
    src = jnp.concatenate(
        [edge_index[0], jnp.zeros((n_pad,), jnp.int32)])
    pad_dst = N_NODES + jnp.arange(n_pad, dtype=jnp.int32) % (NPAD - N_NODES)
    dst = jnp.concatenate([edge_index[1], pad_dst])
    pk = (src | (dst << 16)).reshape(NW, NSB, SB, CHUNK)
    zero = jnp.zeros((NPAD, D), jnp.float32)
    h2 = _segment_sum_sc(inputs, pk, zero)
    return _linear_tc(h2, W, b.reshape(1, D))


# speedup vs baseline: 1.0037x; 1.0036x over previous
import jax
import jax.numpy as jnp
from jax import lax
from jax.experimental import pallas as pl
from jax.experimental.pallas import tpu as pltpu
from jax.experimental.pallas import tpu_sc as plsc

N_NODES = 10000
N_EDGES = 320000
D = 128

NC = 2
NS = 16
NW = NC * NS

CHUNK = 80
NCHUNK = 132
E_PAD = NW * NCHUNK * CHUNK
NPAD = 10112
ROWS_PER_TILE = NPAD // NS


def _scatter_gather_kernel(x_hbm, pk_hbm, zero_hbm, h2_hbm,
                           pk_v, sidx0, sidx1, didx0, didx1,
                           rows0, rows1, acc, sem0, sem1):
    c = lax.axis_index("c")
    s = lax.axis_index("s")
    wid = s * NC + c

    sidx = (sidx0, sidx1)
    didx = (didx0, didx1)
    bufs = (rows0, rows1)
    sems = (sem0, sem1)

    pltpu.sync_copy(pk_hbm.at[wid], pk_v)

    r0 = s * ROWS_PER_TILE
    pltpu.sync_copy(zero_hbm.at[pl.ds(r0, ROWS_PER_TILE)],
                    acc.at[pl.ds(r0, ROWS_PER_TILE)])
    plsc.subcore_barrier()

    def unpack(j, b):
        for t in range(CHUNK // 16):
            v = pk_v[j, pl.ds(t * 16, 16)]
            sidx[b][pl.ds(t * 16, 16)] = jnp.bitwise_and(v, 0xFFFF)
            didx[b][pl.ds(t * 16, 16)] = lax.shift_right_logical(v, 16)

    def gather(b):
        return pltpu.async_copy(x_hbm.at[sidx[b]], bufs[b], sems[b])

    def gather_wait(b):
        pltpu.make_async_copy(x_hbm.at[sidx[b]], bufs[b], sems[b]).wait()

    unpack(0, 0)
    unpack(1, 1)
    gather(0)
    gather(1)

    def body(i2, carry):
        for b in range(2):
            j = i2 * 2 + b
            gather_wait(b)
            pltpu.sync_copy(bufs[b], acc.at[didx[b]], add=True)
            jn = lax.rem(j + 2, NCHUNK)
            unpack(jn, b)
            gather(b)
        return carry

    lax.fori_loop(0, NCHUNK // 2, body, 0)

    gather_wait(0)
    gather_wait(1)

    plsc.subcore_barrier()
    pltpu.sync_copy(acc.at[pl.ds(r0, ROWS_PER_TILE)],
                    h2_hbm.at[c, pl.ds(r0, ROWS_PER_TILE)])


@jax.jit
def _segment_sum_sc(x, pk, zero):
    mesh = plsc.VectorSubcoreMesh(core_axis_name="c", subcore_axis_name="s")
    return pl.kernel(
        _scatter_gather_kernel,
        out_type=jax.ShapeDtypeStruct((NC, NPAD, D), jnp.float32),
        mesh=mesh,
        scratch_types=[
            pltpu.VMEM((NCHUNK, CHUNK), jnp.int32),
            pltpu.VMEM((CHUNK,), jnp.int32),
            pltpu.VMEM((CHUNK,), jnp.int32),
            pltpu.VMEM((CHUNK,), jnp.int32),
            pltpu.VMEM((CHUNK,), jnp.int32),
            pltpu.VMEM((CHUNK, D), jnp.float32),
            pltpu.VMEM((CHUNK, D), jnp.float32),
            pltpu.VMEM_SHARED((NPAD, D), jnp.float32),
            pltpu.SemaphoreType.DMA,
            pltpu.SemaphoreType.DMA,
        ],
    )(x, pk, zero)


def _linear_body(h2_ref, w_ref, b_ref, o_ref):
    h = h2_ref[0] + h2_ref[1]
    o_ref[...] = lax.dot_general(
        h, w_ref[...], (((1,), (1,)), ((), ())),
        preferred_element_type=jnp.float32) + b_ref[...]


@jax.jit
def _linear_tc(h2, W, b2):
    blk = 1000
    grid = N_NODES // blk
    return pl.pallas_call(
        _linear_body,
        grid=(grid,),
        in_specs=[
            pl.BlockSpec((NC, blk, D), lambda i: (0, i, 0)),
            pl.BlockSpec((D, D), lambda i: (0, 0)),
            pl.BlockSpec((1, D), lambda i: (0, 0)),
        ],
        out_specs=pl.BlockSpec((blk, D), lambda i: (i, 0)),
        out_shape=jax.ShapeDtypeStruct((N_NODES, D), jnp.float32),
    )(h2, W, b2)


def kernel(inputs, edge_index, W, b):
    n_pad = E_PAD - N_EDGES
    src = jnp.concatenate(
        [edge_index[0], jnp.zeros((n_pad,), jnp.int32)])
    pad_dst = N_NODES + jnp.arange(n_pad, dtype=jnp.int32) % (NPAD - N_NODES)
    dst = jnp.concatenate([edge_index[1], pad_dst])
    pk = (src | (dst << 16)).reshape(NW, NCHUNK, CHUNK)
    zero = jnp.zeros((NPAD, D), jnp.float32)
    h2 = _segment_sum_sc(inputs, pk, zero)
    return _linear_tc(h2, W, b.reshape(1, D))
